# Initial kernel scaffold; baseline (speedup 1.0000x reference)
import jax
import jax.numpy as jnp
from jax.experimental import pallas as pl


def kernel(input, value_w, value_b, offset_mask_w, offset_mask_b, output_w, output_b):
    raise NotImplementedError("write your pallas kernel here")


# speedup vs baseline: 5786.7266x; 5786.7266x over previous
import functools

import jax
import jax.numpy as jnp
import numpy as np
from jax import lax
from jax.experimental import pallas as pl
from jax.experimental.pallas import tpu as pltpu
from jax.experimental.pallas import tpu_sc as plsc

_N, _H, _W = 8, 32, 32
_L = _H * _W
_CH, _G = 192, 12
_GC = _CH // _G
_P = 9
_OMD = int(np.ceil(_G * _P * 3 / 8) * 8)
_ROWS_A = _CH + _G * 32

_perm = np.zeros((_G * 32,), np.int32)
_keep = np.zeros((_G * 32, 1), np.float32)
for _g in range(_G):
    for _r in range(27):
        if _r < 9:
            _m = 2 * _r
        elif _r < 18:
            _m = 2 * (_r - 9) + 1
        else:
            _m = _r
        _perm[_g * 32 + _r] = _g * 27 + _m
        _keep[_g * 32 + _r, 0] = 1.0


def _proj_body(w_ref, x_ref, b_ref, o_ref):
    o_ref[0] = (
        jnp.dot(w_ref[...], x_ref[0], preferred_element_type=jnp.float32)
        + b_ref[...]
    )


def _proj(w, x, b, rows):
    return pl.pallas_call(
        _proj_body,
        grid=(_N,),
        in_specs=[
            pl.BlockSpec((rows, _CH), lambda n: (0, 0)),
            pl.BlockSpec((1, _CH, _L), lambda n: (n, 0, 0)),
            pl.BlockSpec((rows, 1), lambda n: (0, 0)),
        ],
        out_specs=pl.BlockSpec((1, rows, _L), lambda n: (n, 0, 0)),
        out_shape=jax.ShapeDtypeStruct((_N, rows, _L), jnp.float32),
    )(w, x, b)


_mesh = plsc.VectorSubcoreMesh(core_axis_name="c", subcore_axis_name="s")


@functools.partial(
    pl.kernel,
    mesh=_mesh,
    out_type=jax.ShapeDtypeStruct((_N, _CH, _L), jnp.float32),
    scratch_types=[
        pltpu.VMEM((_GC, _L), jnp.float32),
        pltpu.VMEM((32, _L), jnp.float32),
        pltpu.VMEM((_GC, _L), jnp.float32),
    ],
    compiler_params=pltpu.CompilerParams(
        use_tc_tiling_on_sc=False, needs_layout_passes=False
    ),
)
def _sc_sample(a_hbm, out_hbm, xvt, comp, outv):
    wid = lax.axis_index("s") * 2 + lax.axis_index("c")
    n = wid >> 2
    j = wid & 3

    def tbody(t, carry):
        g = j * 3 + t
        pltpu.sync_copy(a_hbm.at[n, pl.ds(g * _GC, _GC), :], xvt)
        pltpu.sync_copy(a_hbm.at[n, pl.ds(_CH + g * 32, 32), :], comp)

        def bbody(b, c2):
            l0 = b * 16
            li = lax.broadcasted_iota(jnp.int32, (16,), 0) + l0
            pix_y = (li >> 5).astype(jnp.float32)
            pix_x = (li & 31).astype(jnp.float32)
            acc = [jnp.zeros((16,), jnp.float32) for _ in range(_GC)]
            for p in range(_P):
                ky = p // 3 - 1
                kx = p % 3 - 1
                offx = comp[p, pl.ds(l0, 16)]
                offy = comp[9 + p, pl.ds(l0, 16)]
                msk = comp[18 + p, pl.ds(l0, 16)]
                locx = pix_x + (offx + float(kx))
                locy = pix_y + (offy + float(ky))
                xi = locx.astype(jnp.int32)
                yi = locy.astype(jnp.int32)
                xf = xi - jnp.where(locx < xi.astype(jnp.float32), 1, 0)
                yf = yi - jnp.where(locy < yi.astype(jnp.float32), 1, 0)
                lx = locx - xf.astype(jnp.float32)
                ly = locy - yf.astype(jnp.float32)
                hx = 1.0 - lx
                hy = 1.0 - ly
                for dy, dx, bw in (
                    (0, 0, hy * hx),
                    (0, 1, hy * lx),
                    (1, 0, ly * hx),
                    (1, 1, ly * lx),
                ):
                    yy = yf + dy
                    xx = xf + dx
                    valid = (yy >= 0) & (yy < _H) & (xx >= 0) & (xx < _W)
                    yc = jnp.clip(yy, 0, _H - 1)
                    xc = jnp.clip(xx, 0, _W - 1)
                    lin = yc * _W + xc
                    wv = jnp.where(valid, bw * msk, 0.0)
                    for ch in range(_GC):
                        cvec = jnp.full((16,), ch, jnp.int32)
                        val = plsc.load_gather(xvt, [cvec, lin])
                        acc[ch] = acc[ch] + wv * val
            for ch in range(_GC):
                outv[ch, pl.ds(l0, 16)] = acc[ch]
            return c2

        lax.fori_loop(0, _L // 16, bbody, 0)
        pltpu.sync_copy(outv, out_hbm.at[n, pl.ds(g * _GC, _GC), :])
        return carry

    lax.fori_loop(0, 3, tbody, 0)


def kernel(input, value_w, value_b, offset_mask_w, offset_mask_b, output_w, output_b):
    x_t = jnp.transpose(input, (0, 2, 1))
    w2 = offset_mask_w[_perm] * _keep
    b2 = offset_mask_b[_perm] * _keep[:, 0]
    w_all = jnp.concatenate([value_w, w2], axis=0)
    b_all = jnp.concatenate([value_b, b2], axis=0)[:, None]
    a = _proj(w_all, x_t, b_all, _ROWS_A)
    s = _sc_sample(a)
    c = _proj(output_w, s, output_b[:, None], _CH)
    return jnp.transpose(c, (0, 2, 1))
